# baseline (speedup 1.0000x reference)
import functools

import jax
import jax.numpy as jnp
from jax import lax
from jax.experimental import pallas as pl
from jax.experimental.pallas import tpu as pltpu
from jax.experimental.pallas import tpu_sc as plsc

_N = 10000
_E = 320000
_D = 128
_DE = 16

_NPAD = 10240
_RPT = _NPAD // 16
_BLK = 128
_NW = 32
_NBLK = _E // _BLK
_BPW = _NBLK // _NW
_NEXTRA = _NBLK - _BPW * _NW
_BPG = 2
_NGRP = _BPW // _BPG
_K2D = 6
_K2GRP = _BPW // _K2D

_mesh = plsc.VectorSubcoreMesh(core_axis_name="c", subcore_axis_name="s")


def _xsum_body(x_hbm, idxn_hbm, idxd_hbm, xpart_hbm, *scr):
    idxn_ch, idxd_ch = scr[0:2]
    idxn_v = scr[2:4]
    idxd_v = scr[4:6]
    rows_b = scr[6:8]
    zrow_v, xsum = scr[8:10]
    gsem = scr[10:12]
    ssem = scr[12:14]

    c = lax.axis_index("c")
    s = lax.axis_index("s")
    w = s * 2 + c

    zero16 = jnp.zeros((16,), jnp.float32)
    for i in range(8):
        for j in range(_D // 16):
            zrow_v[i, pl.ds(j * 16, 16)] = zero16

    r0 = s * _RPT
    for k in range(_RPT // 8):
        pltpu.sync_copy(zrow_v, xsum.at[pl.ds(r0 + k * 8, 8)])
    plsc.subcore_barrier()

    c0 = 10 * w - jnp.maximum(w - 24, 0)
    nch = jnp.where(w < 24, 10, 9)

    def stage_idx(j, b):
        for t in range(_BLK // 16):
            idxn_v[b][pl.ds(t * 16, 16)] = idxn_ch[j, pl.ds(t * 16, 16)]
            idxd_v[b][pl.ds(t * 16, 16)] = idxd_ch[j, pl.ds(t * 16, 16)]

    def chunk(q, carry):
        cblk = (c0 + q) * 8
        pltpu.sync_copy(idxn_hbm.at[pl.ds(cblk, 8)], idxn_ch)
        pltpu.sync_copy(idxd_hbm.at[pl.ds(cblk, 8)], idxd_ch)
        gds = [None, None]
        sds = [None, None]
        stage_idx(0, 0)
        gds[0] = pltpu.async_copy(x_hbm.at[idxn_v[0]], rows_b[0], gsem[0])
        stage_idx(1, 1)
        gds[1] = pltpu.async_copy(x_hbm.at[idxn_v[1]], rows_b[1], gsem[1])
        gds[0].wait()
        sds[0] = pltpu.async_copy(rows_b[0], xsum.at[idxd_v[0]], ssem[0],
                                  add=True)
        for j in range(2, 8):
            b = j % 2
            sds[b].wait()
            stage_idx(j, b)
            gds[b] = pltpu.async_copy(x_hbm.at[idxn_v[b]], rows_b[b], gsem[b])
            gds[1 - b].wait()
            sds[1 - b] = pltpu.async_copy(rows_b[1 - b],
                                          xsum.at[idxd_v[1 - b]],
                                          ssem[1 - b], add=True)
        gds[1].wait()
        sds[1] = pltpu.async_copy(rows_b[1], xsum.at[idxd_v[1]], ssem[1],
                                  add=True)
        sds[0].wait()
        sds[1].wait()
        return carry

    lax.fori_loop(0, nch, chunk, 0)

    @pl.when(w == _NW - 1)
    def _():
        pltpu.sync_copy(idxn_hbm.at[pl.ds(312 * 8, 4)],
                        idxn_ch.at[pl.ds(0, 4)])
        pltpu.sync_copy(idxd_hbm.at[pl.ds(312 * 8, 4)],
                        idxd_ch.at[pl.ds(0, 4)])
        for j in range(4):
            stage_idx(j, 0)
            pltpu.sync_copy(x_hbm.at[idxn_v[0]], rows_b[0])
            pltpu.sync_copy(rows_b[0], xsum.at[idxd_v[0]], add=True)

    plsc.subcore_barrier()

    pltpu.sync_copy(xsum.at[pl.ds(r0, _RPT)],
                    xpart_hbm.at[c].at[pl.ds(r0, _RPT)])


_xsum_call = functools.partial(
    pl.kernel,
    out_type=jax.ShapeDtypeStruct((2, _NPAD, _D), jnp.float32),
    mesh=_mesh,
    scratch_types=(
        [pltpu.VMEM((8, _BLK), jnp.int32) for _ in range(2)]
        + [pltpu.VMEM((_BLK,), jnp.int32) for _ in range(4)]
        + [pltpu.VMEM((_BLK, _D), jnp.float32) for _ in range(2)]
        + [
            pltpu.VMEM((8, _D), jnp.float32),
            pltpu.VMEM_SHARED((_NPAD, _D), jnp.float32),
        ]
        + [pltpu.SemaphoreType.DMA for _ in range(4)]
    ),
)(_xsum_body)


_RPW = _NPAD // _NW
_ACC = _RPW * (_DE + 1)
_PADE = 384


def _ef_body(idxd_hbm, ef_hbm, eout_hbm, *scr):
    idxd_b = scr[0:2]
    ef_b = scr[2:4]
    pb, acc = scr[4:6]
    lsem = scr[6:8]

    c = lax.axis_index("c")
    s = lax.axis_index("s")
    w = s * 2 + c

    zero16 = jnp.zeros((16,), jnp.float32)
    one16 = jnp.ones((16,), jnp.float32)
    iota16 = lax.iota(jnp.int32, 16)

    def zero_acc(t, carry):
        acc[pl.ds(pl.multiple_of(t * 16, 16), 16)] = zero16
        return carry

    lax.fori_loop(0, (_ACC + 16) // 16, zero_acc, 0)

    wlo = w * _RPW
    whi = wlo + _RPW

    def lower_bound(v):
        def it(i, lh):
            lo, hi = lh
            mid = (lo + hi) // 2
            m16 = pl.multiple_of((mid // 16) * 16, 16)
            pltpu.sync_copy(idxd_hbm.at[pl.ds(m16, 16)], pb)
            probe = pb[pl.ds(0, 16)]
            val = jnp.max(jnp.where(iota16 == mid - m16, probe,
                                    jnp.int32(-2147483647 - 1)))
            pred = val < v
            return (jnp.where(pred, mid + 1, lo), jnp.where(pred, hi, mid))

        lo, _hi = lax.fori_loop(0, 19, it, (jnp.int32(0), jnp.int32(_E)))
        return lo

    lo = lower_bound(wlo)
    hi = lower_bound(whi)
    e0 = pl.multiple_of((lo // 8) * 8, 8)
    npair = (hi - e0 + 2 * _BLK - 1) // (2 * _BLK)

    def load_block(k, b):
        off = pl.multiple_of(e0 + k * _BLK, 8)
        da = pltpu.async_copy(idxd_hbm.at[pl.ds(off, _BLK)], idxd_b[b],
                              lsem[b])
        db = pltpu.async_copy(ef_hbm.at[pl.ds(off * _DE, _BLK * _DE)],
                              ef_b[b], lsem[b])
        return da, db

    def drain(b):
        pltpu.make_async_copy(idxd_hbm.at[pl.ds(0, _BLK)], idxd_b[b],
                              lsem[b]).wait()
        pltpu.make_async_copy(ef_hbm.at[pl.ds(0, _BLK * _DE)], ef_b[b],
                              lsem[b]).wait()

    def compute(b):
        for g in range(_BLK // 16):
            i16 = idxd_b[b][pl.ds(g * 16, 16)]
            inb = (i16 >= wlo) & (i16 < whi)
            a16 = jnp.where(inb, (i16 - wlo) * (_DE + 1), _ACC)
            pos = iota16 * _DE + g * 16 * _DE
            for col in range(_DE):
                v16 = plsc.load_gather(ef_b[b], [pos + col])
                plsc.addupdate_scatter(acc, [a16 + col], v16)
            plsc.addupdate_scatter(acc, [a16 + _DE], one16)

    @pl.when(npair > 0)
    def _():
        load_block(0, 0)

        def pair(p, carry):
            load_block(2 * p + 1, 1)
            drain(0)
            compute(0)

            @pl.when(p + 1 < npair)
            def _():
                load_block(2 * p + 2, 0)

            drain(1)
            compute(1)
            return carry

        lax.fori_loop(0, npair, pair, 0)

    pltpu.sync_copy(acc.at[pl.ds(0, _ACC)],
                    eout_hbm.at[pl.ds(w * _ACC, _ACC)])


_ef_call = functools.partial(
    pl.kernel,
    out_type=jax.ShapeDtypeStruct((_NPAD * (_DE + 1),), jnp.float32),
    mesh=_mesh,
    compiler_params=pltpu.CompilerParams(needs_layout_passes=False),
    scratch_types=(
        [pltpu.VMEM((_BLK,), jnp.int32) for _ in range(2)]
        + [pltpu.VMEM((_BLK * _DE,), jnp.float32) for _ in range(2)]
        + [
            pltpu.VMEM((16,), jnp.int32),
            pltpu.VMEM((_ACC + 16,), jnp.float32),
        ]
        + [pltpu.SemaphoreType.DMA for _ in range(2)]
    ),
)(_ef_body)


_BN = 1000


def _tc_body(x_ref, x0_ref, x1_ref, e_ref, o_ref):
    e = e_ref[...]
    deg = e[:, _DE:_DE + 1]
    inv = 1.0 / jnp.maximum(deg, 1.0)
    mask = (deg > 0.0).astype(jnp.float32)
    xs = (x0_ref[0] + x1_ref[0]) * inv
    es = e[:, 0:_DE] * inv
    xm = x_ref[...] * mask
    o_ref[...] = jnp.concatenate([xs, xm, es], axis=1)


def _tc_finalize(x, xpart, e17):
    return pl.pallas_call(
        _tc_body,
        grid=(_N // _BN,),
        in_specs=[
            pl.BlockSpec((_BN, _D), lambda r: (r, 0)),
            pl.BlockSpec((1, _BN, _D), lambda r: (0, r, 0)),
            pl.BlockSpec((1, _BN, _D), lambda r: (1, r, 0)),
            pl.BlockSpec((_BN, _DE + 1), lambda r: (r, 0)),
        ],
        out_specs=pl.BlockSpec((_BN, 2 * _D + _DE), lambda r: (r, 0)),
        out_shape=jax.ShapeDtypeStruct((_N, 2 * _D + _DE), jnp.float32),
    )(x, xpart, xpart, e17)


def kernel(x, idxn, idxd, edgefeats):
    idxn2 = idxn.reshape(_NBLK, _BLK)
    idxd2 = idxd.reshape(_NBLK, _BLK)
    xpart = _xsum_call(x, idxn2, idxd2)
    idxd_pad = jnp.concatenate(
        [idxd, jnp.full((_PADE,), _NPAD, jnp.int32)])
    ef_pad = jnp.concatenate(
        [edgefeats.reshape(-1), jnp.zeros((_PADE * _DE,), jnp.float32)])
    e17 = _ef_call(idxd_pad, ef_pad).reshape(_NPAD, _DE + 1)
    return _tc_finalize(x, xpart, e17)


# speedup vs baseline: 1.6921x; 1.6921x over previous
import functools

import jax
import jax.numpy as jnp
from jax import lax
from jax.experimental import pallas as pl
from jax.experimental.pallas import tpu as pltpu
from jax.experimental.pallas import tpu_sc as plsc

_N = 10000
_E = 320000
_D = 128
_DE = 16

_NPAD = 10240
_RPT = _NPAD // 16
_BLK = 128
_NW = 32
_NBLK = _E // _BLK
_BPW = _NBLK // _NW
_NEXTRA = _NBLK - _BPW * _NW
_BPG = 2
_NGRP = _BPW // _BPG
_K2D = 6
_K2GRP = _BPW // _K2D

_mesh = plsc.VectorSubcoreMesh(core_axis_name="c", subcore_axis_name="s")


def _xsum_body(x_hbm, idxn_hbm, idxd_hbm, xpart_hbm, *scr):
    idxn_ch, idxd_ch = scr[0:2]
    idxn_v = scr[2:4]
    idxd_v = scr[4:6]
    rows_b = scr[6:8]
    zrow_v, xsum = scr[8:10]
    gsem = scr[10:12]
    ssem = scr[12:14]

    c = lax.axis_index("c")
    s = lax.axis_index("s")
    w = s * 2 + c

    zero16 = jnp.zeros((16,), jnp.float32)
    for i in range(8):
        for j in range(_D // 16):
            zrow_v[i, pl.ds(j * 16, 16)] = zero16

    r0 = s * _RPT
    for k in range(_RPT // 8):
        pltpu.sync_copy(zrow_v, xsum.at[pl.ds(r0 + k * 8, 8)])
    plsc.subcore_barrier()

    c0 = 10 * w - jnp.maximum(w - 24, 0)
    nch = jnp.where(w < 24, 10, 9)

    def stage_idx(j, b):
        for t in range(_BLK // 16):
            idxn_v[b][pl.ds(t * 16, 16)] = idxn_ch[j, pl.ds(t * 16, 16)]
            idxd_v[b][pl.ds(t * 16, 16)] = idxd_ch[j, pl.ds(t * 16, 16)]

    def chunk(q, carry):
        cblk = (c0 + q) * 8
        pltpu.sync_copy(idxn_hbm.at[pl.ds(cblk, 8)], idxn_ch)
        pltpu.sync_copy(idxd_hbm.at[pl.ds(cblk, 8)], idxd_ch)
        gds = [None, None]
        sds = [None, None]
        stage_idx(0, 0)
        gds[0] = pltpu.async_copy(x_hbm.at[idxn_v[0]], rows_b[0], gsem[0])
        stage_idx(1, 1)
        gds[1] = pltpu.async_copy(x_hbm.at[idxn_v[1]], rows_b[1], gsem[1])
        gds[0].wait()
        sds[0] = pltpu.async_copy(rows_b[0], xsum.at[idxd_v[0]], ssem[0],
                                  add=True)
        for j in range(2, 8):
            b = j % 2
            sds[b].wait()
            stage_idx(j, b)
            gds[b] = pltpu.async_copy(x_hbm.at[idxn_v[b]], rows_b[b], gsem[b])
            gds[1 - b].wait()
            sds[1 - b] = pltpu.async_copy(rows_b[1 - b],
                                          xsum.at[idxd_v[1 - b]],
                                          ssem[1 - b], add=True)
        gds[1].wait()
        sds[1] = pltpu.async_copy(rows_b[1], xsum.at[idxd_v[1]], ssem[1],
                                  add=True)
        sds[0].wait()
        sds[1].wait()
        return carry

    lax.fori_loop(0, nch, chunk, 0)

    @pl.when(w == _NW - 1)
    def _():
        pltpu.sync_copy(idxn_hbm.at[pl.ds(312 * 8, 4)],
                        idxn_ch.at[pl.ds(0, 4)])
        pltpu.sync_copy(idxd_hbm.at[pl.ds(312 * 8, 4)],
                        idxd_ch.at[pl.ds(0, 4)])
        for j in range(4):
            stage_idx(j, 0)
            pltpu.sync_copy(x_hbm.at[idxn_v[0]], rows_b[0])
            pltpu.sync_copy(rows_b[0], xsum.at[idxd_v[0]], add=True)

    plsc.subcore_barrier()

    pltpu.sync_copy(xsum.at[pl.ds(r0, _RPT)],
                    xpart_hbm.at[c].at[pl.ds(r0, _RPT)])


_xsum_call = functools.partial(
    pl.kernel,
    out_type=jax.ShapeDtypeStruct((2, _NPAD, _D), jnp.float32),
    mesh=_mesh,
    scratch_types=(
        [pltpu.VMEM((8, _BLK), jnp.int32) for _ in range(2)]
        + [pltpu.VMEM((_BLK,), jnp.int32) for _ in range(4)]
        + [pltpu.VMEM((_BLK, _D), jnp.float32) for _ in range(2)]
        + [
            pltpu.VMEM((8, _D), jnp.float32),
            pltpu.VMEM_SHARED((_NPAD, _D), jnp.float32),
        ]
        + [pltpu.SemaphoreType.DMA for _ in range(4)]
    ),
)(_xsum_body)


_RPW = _NPAD // _NW
_RSTR = 128
_ACC = _RPW * _RSTR
_PADE = 384


def _ef_body(idxd_hbm, ef_hbm, eout_hbm, *scr):
    idxd_b = scr[0:2]
    ef_b = scr[2:4]
    pb, acc = scr[4:6]
    lsem = scr[6:8]

    c = lax.axis_index("c")
    s = lax.axis_index("s")
    w = s * 2 + c

    zero16 = jnp.zeros((16,), jnp.float32)
    one16 = jnp.ones((16,), jnp.float32)
    iota16 = lax.iota(jnp.int32, 16)

    def zero_acc(t, carry):
        acc[pl.ds(pl.multiple_of(t * 16, 16), 16)] = zero16
        return carry

    lax.fori_loop(0, (_ACC + 16) // 16, zero_acc, 0)

    wlo = w * _RPW
    whi = wlo + _RPW

    def lower_bound(v):
        def it(i, lh):
            lo, hi = lh
            mid = (lo + hi) // 2
            m16 = pl.multiple_of((mid // 16) * 16, 16)
            pltpu.sync_copy(idxd_hbm.at[pl.ds(m16, 16)], pb)
            probe = pb[pl.ds(0, 16)]
            val = jnp.max(jnp.where(iota16 == mid - m16, probe,
                                    jnp.int32(-2147483647 - 1)))
            pred = val < v
            return (jnp.where(pred, mid + 1, lo), jnp.where(pred, hi, mid))

        lo, _hi = lax.fori_loop(0, 19, it, (jnp.int32(0), jnp.int32(_E)))
        return lo

    lo = lower_bound(wlo)
    hi = lower_bound(whi)
    e0 = pl.multiple_of((lo // 8) * 8, 8)
    npair = (hi - e0 + 2 * _BLK - 1) // (2 * _BLK)

    def load_block(k, b):
        off = pl.multiple_of(e0 + k * _BLK, 8)
        pltpu.async_copy(idxd_hbm.at[pl.ds(off, _BLK)], idxd_b[b], lsem[b])
        pltpu.async_copy(ef_hbm.at[pl.ds(off * _DE, _BLK * _DE)], ef_b[b],
                         lsem[b])

    def drain(b):
        pltpu.make_async_copy(idxd_hbm.at[pl.ds(0, _BLK)], idxd_b[b],
                              lsem[b]).wait()
        pltpu.make_async_copy(ef_hbm.at[pl.ds(0, _BLK * _DE)], ef_b[b],
                              lsem[b]).wait()

    def compute(b):
        for g in range(_BLK // 16):
            i16 = idxd_b[b][pl.ds(g * 16, 16)]
            inb = (i16 >= wlo) & (i16 < whi)
            a16 = jnp.where(inb, (i16 - wlo) * _RSTR, _ACC)
            pb[pl.ds(0, 16)] = a16
            for l in range(16):
                al = plsc.load_gather(pb, [jnp.full((16,), l, jnp.int32)])
                v16 = ef_b[b][pl.ds((g * 16 + l) * _DE, _DE)]
                plsc.addupdate_scatter(acc, [al + iota16], v16)
            plsc.addupdate_scatter(acc, [a16 + _DE], one16)

    @pl.when(npair > 0)
    def _():
        load_block(0, 0)

        def pair(p, carry):
            load_block(2 * p + 1, 1)
            drain(0)
            compute(0)

            @pl.when(p + 1 < npair)
            def _():
                load_block(2 * p + 2, 0)

            drain(1)
            compute(1)
            return carry

        lax.fori_loop(0, npair, pair, 0)

    pltpu.sync_copy(acc.at[pl.ds(0, _ACC)],
                    eout_hbm.at[pl.ds(w * _ACC, _ACC)])


_ef_call = functools.partial(
    pl.kernel,
    out_type=jax.ShapeDtypeStruct((_NPAD * _RSTR,), jnp.float32),
    mesh=_mesh,
    compiler_params=pltpu.CompilerParams(needs_layout_passes=False),
    scratch_types=(
        [pltpu.VMEM((_BLK,), jnp.int32) for _ in range(2)]
        + [pltpu.VMEM((_BLK * _DE,), jnp.float32) for _ in range(2)]
        + [
            pltpu.VMEM((16,), jnp.int32),
            pltpu.VMEM((_ACC + 16,), jnp.float32),
        ]
        + [pltpu.SemaphoreType.DMA for _ in range(2)]
    ),
)(_ef_body)


_BN = 1000


def _tc_body(x_ref, x0_ref, x1_ref, e_ref, o_ref):
    e = e_ref[...]
    deg = e[:, _DE:_DE + 1]
    inv = 1.0 / jnp.maximum(deg, 1.0)
    mask = (deg > 0.0).astype(jnp.float32)
    xs = (x0_ref[0] + x1_ref[0]) * inv
    es = e[:, 0:_DE] * inv
    xm = x_ref[...] * mask
    o_ref[...] = jnp.concatenate([xs, xm, es], axis=1)


def _tc_finalize(x, xpart, e17):
    return pl.pallas_call(
        _tc_body,
        grid=(_N // _BN,),
        in_specs=[
            pl.BlockSpec((_BN, _D), lambda r: (r, 0)),
            pl.BlockSpec((1, _BN, _D), lambda r: (0, r, 0)),
            pl.BlockSpec((1, _BN, _D), lambda r: (1, r, 0)),
            pl.BlockSpec((_BN, _RSTR), lambda r: (r, 0)),
        ],
        out_specs=pl.BlockSpec((_BN, 2 * _D + _DE), lambda r: (r, 0)),
        out_shape=jax.ShapeDtypeStruct((_N, 2 * _D + _DE), jnp.float32),
    )(x, xpart, xpart, e17)


def kernel(x, idxn, idxd, edgefeats):
    idxn2 = idxn.reshape(_NBLK, _BLK)
    idxd2 = idxd.reshape(_NBLK, _BLK)
    xpart = _xsum_call(x, idxn2, idxd2)
    idxd_pad = jnp.concatenate(
        [idxd, jnp.full((_PADE,), _NPAD, jnp.int32)])
    ef_pad = jnp.concatenate(
        [edgefeats.reshape(-1), jnp.zeros((_PADE * _DE,), jnp.float32)])
    e17 = _ef_call(idxd_pad, ef_pad).reshape(_NPAD, _RSTR)
    return _tc_finalize(x, xpart, e17)
